# Initial kernel scaffold; baseline (speedup 1.0000x reference)
import jax
import jax.numpy as jnp
from jax.experimental import pallas as pl


def kernel(edge_attr, edge_index, batch, W1, b1, W2, b2, W3, b3, Wc1, bc1, Wc2, bc2):
    raise NotImplementedError("write your pallas kernel here")


# speedup vs baseline: 5.0748x; 5.0748x over previous
import functools

import jax
import jax.numpy as jnp
from jax import lax
from jax.experimental import pallas as pl
from jax.experimental.pallas import tpu as pltpu
from jax.experimental.pallas import tpu_sc as plsc

NN = 10000
NP = 10240
NE = 160000
ED = 16
HID = 256
HH = 128
NG = 16
NC = 2
NS = 16

EA = NE // (NC * NS)
CHA = 40
NCHA = EA // CHA

ET = NE // NS
CH = 80
NCH = ET // CH

RT = NP // NS
RC = 64
NRC = RT // RC

def _mesh():
    return plsc.VectorSubcoreMesh(core_axis_name="c", subcore_axis_name="s",
                                  num_cores=NC, num_subcores=NS)


NEG = -3.4e38


def _ka_body(attr_hbm, dst_hbm, z_hbm, pab_hbm, acc, abuf, sbuf, ibuf):
    c = lax.axis_index("c")
    s = lax.axis_index("s")
    r0 = s * RT
    pltpu.sync_copy(z_hbm, acc.at[pl.ds(r0, RT)])

    ones = jnp.ones((16,), jnp.float32)
    zeros = jnp.zeros((16,), jnp.float32)

    def preset(i, carry):
        sbuf[i, pl.ds(16, 16)] = ones
        for g in range(2, HH // 16):
            sbuf[i, pl.ds(g * 16, 16)] = zeros
        return carry

    lax.fori_loop(0, CHA, preset, 0)
    plsc.subcore_barrier()

    base = (c * NS + s) * EA

    def chunk(i, carry):
        e0 = base + i * CHA
        pltpu.sync_copy(dst_hbm.at[pl.ds(e0, CHA)], ibuf)
        pltpu.sync_copy(attr_hbm.at[pl.ds(e0, CHA), :], abuf)

        def stage(k, carry2):
            sbuf[k, pl.ds(0, 16)] = abuf[k, :]
            return carry2

        lax.fori_loop(0, CHA, stage, 0)
        pltpu.sync_copy(sbuf, acc.at[ibuf], add=True)
        return carry

    lax.fori_loop(0, NCHA, chunk, 0)
    plsc.subcore_barrier()
    pltpu.sync_copy(acc.at[pl.ds(r0, RT)],
                    pab_hbm.at[pl.ds(c * NP + r0, RT), :])


def _run_ka(edge_attr, dst, zeros_r128):
    f = pl.kernel(
        _ka_body,
        out_type=jax.ShapeDtypeStruct((2 * NP, HH), jnp.float32),
        mesh=_mesh(),
        scratch_types=[
            pltpu.VMEM_SHARED((NP, HH), jnp.float32),
            pltpu.VMEM((CHA, ED), jnp.float32),
            pltpu.VMEM((CHA, HH), jnp.float32),
            pltpu.VMEM((CHA,), jnp.int32),
        ],
    )
    return f(edge_attr, dst, zeros_r128)


def _kc_body(em_hbm, eml_hbm, dst_hbm, s_hbm, x_hbm, acc, ebuf, ibuf, tbuf):
    c = lax.axis_index("c")
    s = lax.axis_index("s")
    col0 = c * HH
    r0 = s * RT
    pltpu.sync_copy(eml_hbm.at[pl.ds(r0, RT), pl.ds(col0, HH)],
                    acc.at[pl.ds(r0, RT)])
    plsc.subcore_barrier()

    base = s * ET

    def chunk(i, carry):
        e0 = base + i * CH
        pltpu.sync_copy(dst_hbm.at[pl.ds(e0, CH)], ibuf)
        pltpu.sync_copy(em_hbm.at[pl.ds(e0, CH), pl.ds(col0, HH)], ebuf)
        pltpu.sync_copy(ebuf, acc.at[ibuf], add=True)
        return carry

    lax.fori_loop(0, NCH, chunk, 0)
    plsc.subcore_barrier()

    def out_chunk(j, carry):
        rr = r0 + j * RC
        pltpu.sync_copy(acc.at[pl.ds(rr, RC)], tbuf)
        pltpu.sync_copy(tbuf, s_hbm.at[pl.ds(rr, RC), pl.ds(col0, HH)])

        def relu_row(i, carry2):
            for g in range(HH // 16):
                v = tbuf[i, pl.ds(g * 16, 16)]
                tbuf[i, pl.ds(g * 16, 16)] = jnp.maximum(v, 0.0)
            return carry2

        lax.fori_loop(0, RC, relu_row, 0)
        pltpu.sync_copy(tbuf, x_hbm.at[pl.ds(c * NP + rr, RC), :])
        return carry

    lax.fori_loop(0, NRC, out_chunk, 0)


def _run_kc(em, eml, dst):
    f = pl.kernel(
        _kc_body,
        out_type=[jax.ShapeDtypeStruct((NP, HID), jnp.float32),
                  jax.ShapeDtypeStruct((2 * NP, HH), jnp.float32)],
        mesh=_mesh(),
        scratch_types=[
            pltpu.VMEM_SHARED((NP, HH), jnp.float32),
            pltpu.VMEM((CH, HH), jnp.float32),
            pltpu.VMEM((CH,), jnp.int32),
            pltpu.VMEM((RC, HH), jnp.float32),
        ],
    )
    return f(em, eml, dst)


def _kd_body(xprev_hbm, s_hbm, src_hbm, dst_hbm, z_hbm, xnext_hbm,
             acc, gbuf, sibuf, dibuf, tbuf, tbuf2, tbuf3, sem):
    c = lax.axis_index("c")
    s = lax.axis_index("s")
    col0 = c * HH
    r0 = s * RT
    roff = c * NP
    pltpu.sync_copy(z_hbm, acc.at[pl.ds(r0, RT)])
    plsc.subcore_barrier()

    base = s * ET

    def chunk(i, carry):
        e0 = base + i * CH
        pltpu.sync_copy(src_hbm.at[pl.ds(e0, CH)], sibuf)
        for k in range(CH // 16):
            sibuf[pl.ds(k * 16, 16)] = sibuf[pl.ds(k * 16, 16)] + roff
        pltpu.async_copy(xprev_hbm.at[sibuf], gbuf, sem).wait()
        pltpu.sync_copy(dst_hbm.at[pl.ds(e0, CH)], dibuf)
        pltpu.sync_copy(gbuf, acc.at[dibuf], add=True)
        return carry

    lax.fori_loop(0, NCH, chunk, 0)
    plsc.subcore_barrier()

    def out_chunk(j, carry):
        rr = r0 + j * RC
        pltpu.sync_copy(acc.at[pl.ds(rr, RC)], tbuf)
        pltpu.sync_copy(s_hbm.at[pl.ds(rr, RC), pl.ds(col0, HH)], tbuf2)
        pltpu.sync_copy(xprev_hbm.at[pl.ds(roff + rr, RC), :], tbuf3)

        def fuse_row(i, carry2):
            for g in range(HH // 16):
                sl = pl.ds(g * 16, 16)
                v = tbuf[i, sl] + tbuf2[i, sl] + tbuf3[i, sl]
                tbuf[i, sl] = jnp.maximum(v, 0.0)
            return carry2

        lax.fori_loop(0, RC, fuse_row, 0)
        pltpu.sync_copy(tbuf, xnext_hbm.at[pl.ds(roff + rr, RC), :])
        return carry

    lax.fori_loop(0, NRC, out_chunk, 0)


def _run_kd(xprev, s_arr, src, dst, zeros_r128):
    f = pl.kernel(
        _kd_body,
        out_type=jax.ShapeDtypeStruct((2 * NP, HH), jnp.float32),
        mesh=_mesh(),
        scratch_types=[
            pltpu.VMEM_SHARED((NP, HH), jnp.float32),
            pltpu.VMEM((CH, HH), jnp.float32),
            pltpu.VMEM((CH,), jnp.int32),
            pltpu.VMEM((CH,), jnp.int32),
            pltpu.VMEM((RC, HH), jnp.float32),
            pltpu.VMEM((RC, HH), jnp.float32),
            pltpu.VMEM((RC, HH), jnp.float32),
            pltpu.SemaphoreType.DMA,
        ],
    )
    return f(xprev, s_arr, src, dst, zeros_r128)


def _kb1_body(attr_ref, w1_ref, b1_ref, w2_ref, b2_ref, w3_ref, b3_ref,
              out_ref):
    a = attr_ref[...]
    h = jnp.dot(a, w1_ref[...], preferred_element_type=jnp.float32)
    h = jnp.maximum(h + b1_ref[...], 0.0)
    h = jnp.dot(h, w2_ref[...], preferred_element_type=jnp.float32)
    h = jnp.maximum(h + b2_ref[...], 0.0)
    h = jnp.dot(h, w3_ref[...], preferred_element_type=jnp.float32)
    out_ref[...] = h + b3_ref[...]


def _run_kb1(edge_attr, w1, b1, w2, b2, w3, b3):
    blk = 1280
    nblk = NE // blk
    full = lambda shape: pl.BlockSpec(shape, lambda i: (0, 0))
    return pl.pallas_call(
        _kb1_body,
        grid=(nblk,),
        in_specs=[
            pl.BlockSpec((blk, ED), lambda i: (i, 0)),
            full((ED, HID)), full((1, HID)),
            full((HID, HID)), full((1, HID)),
            full((HID, HID)), full((1, HID)),
        ],
        out_specs=pl.BlockSpec((blk, HID), lambda i: (i, 0)),
        out_shape=jax.ShapeDtypeStruct((NE, HID), jnp.float32),
    )(edge_attr, w1, b1, w2, b2, w3, b3)


def _kb2_body(p0_ref, p1_ref,
              w1_ref, b1_ref, w2_ref, b2_ref, w3_ref, b3_ref, out_ref):
    psum = p0_ref[...] + p1_ref[...]
    attr_sum = psum[:, 0:ED]
    deg = psum[:, ED:ED + 1]
    la = attr_sum / jnp.maximum(deg, 1.0)
    h = jnp.dot(la, w1_ref[...], preferred_element_type=jnp.float32)
    h = jnp.maximum(h + b1_ref[...], 0.0)
    h = jnp.dot(h, w2_ref[...], preferred_element_type=jnp.float32)
    h = jnp.maximum(h + b2_ref[...], 0.0)
    h = jnp.dot(h, w3_ref[...], preferred_element_type=jnp.float32)
    out_ref[...] = h + b3_ref[...]


def _run_kb2(pab, w1, b1, w2, b2, w3, b3):
    blk = 1024
    nblk = NP // blk
    full = lambda shape: pl.BlockSpec(shape, lambda i: (0, 0))
    return pl.pallas_call(
        _kb2_body,
        grid=(nblk,),
        in_specs=[
            pl.BlockSpec((blk, HH), lambda i: (i, 0)),
            pl.BlockSpec((blk, HH), lambda i: (i + nblk, 0)),
            full((ED, HID)), full((1, HID)),
            full((HID, HID)), full((1, HID)),
            full((HID, HID)), full((1, HID)),
        ],
        out_specs=pl.BlockSpec((blk, HID), lambda i: (i, 0)),
        out_shape=jax.ShapeDtypeStruct((NP, HID), jnp.float32),
    )(pab, pab, w1, b1, w2, b2, w3, b3)


def _ke_body(xa_ref, xb_ref, batch_ref, wc1_ref, bc1_ref, wc2_ref, bc2_ref,
             out_ref, g_scr):
    i = pl.program_id(0)
    nsteps = pl.num_programs(0)

    @pl.when(i == 0)
    def _():
        g_scr[...] = jnp.full((NG, HID), NEG, jnp.float32)

    b = batch_ref[...]
    xa = xa_ref[...]
    xb = xb_ref[...]
    for j in range(NG):
        m = b == j
        va = jnp.max(jnp.where(m, xa, NEG), axis=0, keepdims=True)
        vb = jnp.max(jnp.where(m, xb, NEG), axis=0, keepdims=True)
        g_scr[j:j + 1, 0:HH] = jnp.maximum(g_scr[j:j + 1, 0:HH], va)
        g_scr[j:j + 1, HH:HID] = jnp.maximum(g_scr[j:j + 1, HH:HID], vb)

    @pl.when(i == nsteps - 1)
    def _():
        g = g_scr[...]
        h = jnp.dot(g, wc1_ref[...], preferred_element_type=jnp.float32)
        h = jnp.maximum(h + bc1_ref[...], 0.0)
        h = jnp.dot(h, wc2_ref[...], preferred_element_type=jnp.float32)
        out_ref[...] = h + bc2_ref[...]


def _run_ke(x3, batch2, wc1, bc1, wc2, bc2):
    blk = 1024
    nblk = NP // blk
    full = lambda shape: pl.BlockSpec(shape, lambda i: (0, 0))
    ncls = wc2.shape[1]
    return pl.pallas_call(
        _ke_body,
        grid=(nblk,),
        in_specs=[
            pl.BlockSpec((blk, HH), lambda i: (i, 0)),
            pl.BlockSpec((blk, HH), lambda i: (i + nblk, 0)),
            pl.BlockSpec((blk, 1), lambda i: (i, 0)),
            full((HID, HID)), full((1, HID)),
            full((HID, ncls)), full((1, ncls)),
        ],
        out_specs=pl.BlockSpec((NG, ncls), lambda i: (0, 0)),
        out_shape=jax.ShapeDtypeStruct((NG, ncls), jnp.float32),
        scratch_shapes=[pltpu.VMEM((NG, HID), jnp.float32)],
    )(x3, x3, batch2, wc1, bc1, wc2, bc2)


def kernel(edge_attr, edge_index, batch, W1, b1, W2, b2, W3, b3,
           Wc1, bc1, Wc2, bc2):
    src = edge_index[0].astype(jnp.int32)
    dst = edge_index[1].astype(jnp.int32)
    batch_pad = jnp.concatenate(
        [batch.astype(jnp.int32), jnp.full((NP - NN,), NG, jnp.int32)])
    batch2 = batch_pad.reshape(NP, 1)
    zeros_r128 = jnp.zeros((RT, HH), jnp.float32)
    b1r = b1.reshape(1, HID)
    b2r = b2.reshape(1, HID)
    b3r = b3.reshape(1, HID)
    bc1r = bc1.reshape(1, HID)
    bc2r = bc2.reshape(1, -1)

    pab = _run_ka(edge_attr, dst, zeros_r128)
    em = _run_kb1(edge_attr, W1, b1r, W2, b2r, W3, b3r)
    eml = _run_kb2(pab, W1, b1r, W2, b2r, W3, b3r)
    s_arr, x1 = _run_kc(em, eml, dst)
    x2 = _run_kd(x1, s_arr, src, dst, zeros_r128)
    x3 = _run_kd(x2, s_arr, src, dst, zeros_r128)
    return _run_ke(x3, batch2, Wc1, bc1r, Wc2, bc2r)


# speedup vs baseline: 5.9319x; 1.1689x over previous
import functools

import jax
import jax.numpy as jnp
from jax import lax
from jax.experimental import pallas as pl
from jax.experimental.pallas import tpu as pltpu
from jax.experimental.pallas import tpu_sc as plsc

NN = 10000
NP = 10240
NE = 160000
ED = 16
HID = 256
HH = 128
NG = 16
NC = 2
NS = 16

EA = NE // (NC * NS)
CHA = 40
NCHA = EA // CHA

ET = NE // NS
CH = 80
NCH = ET // CH

RT = NP // NS
RC = 64
NRC = RT // RC

def _mesh():
    return plsc.VectorSubcoreMesh(core_axis_name="c", subcore_axis_name="s",
                                  num_cores=NC, num_subcores=NS)


NEG = -3.4e38


def _ka_body(attr_hbm, dst_hbm, z_hbm, pab_hbm, acc,
             ab0, ab1, sb0, sb1, ib0, ib1, sc0, sc1):
    c = lax.axis_index("c")
    s = lax.axis_index("s")
    r0 = s * RT
    pltpu.sync_copy(z_hbm, acc.at[pl.ds(r0, RT)])

    ones = jnp.ones((16,), jnp.float32)
    abs_ = (ab0, ab1)
    sbs = (sb0, sb1)
    ibs = (ib0, ib1)
    scs = (sc0, sc1)

    def preset(i, carry):
        for b in range(2):
            sbs[b][i, pl.ds(16, 16)] = ones
        return carry

    lax.fori_loop(0, CHA, preset, 0)
    plsc.subcore_barrier()

    base = (c * NS + s) * EA

    def do_chunk(j, b, guarded):
        e0 = base + j * CHA
        if guarded:
            @pl.when(j >= 2)
            def _():
                pltpu.make_async_copy(sbs[b], acc.at[ibs[b]], scs[b]).wait()
        else:
            pltpu.make_async_copy(sbs[b], acc.at[ibs[b]], scs[b]).wait()
        pltpu.sync_copy(dst_hbm.at[pl.ds(e0, CHA)], ibs[b])
        pltpu.sync_copy(attr_hbm.at[pl.ds(e0, CHA), :], abs_[b])
        for k in range(CHA):
            sbs[b][k, pl.ds(0, 16)] = abs_[b][k, :]
        pltpu.async_copy(sbs[b], acc.at[ibs[b]], scs[b], add=True)

    def pair(i, carry):
        for b in range(2):
            do_chunk(2 * i + b, b, True)
        return carry

    lax.fori_loop(0, NCHA // 2, pair, 0)
    do_chunk(NCHA - 1, 0, False)
    pltpu.make_async_copy(sb0, acc.at[ib0], sc0).wait()
    pltpu.make_async_copy(sb1, acc.at[ib1], sc1).wait()
    plsc.subcore_barrier()
    pltpu.sync_copy(acc.at[pl.ds(r0, RT)],
                    pab_hbm.at[pl.ds(c * NP + r0, RT), :])


def _run_ka(edge_attr, dst, zeros_r128):
    f = pl.kernel(
        _ka_body,
        out_type=jax.ShapeDtypeStruct((2 * NP, HH), jnp.float32),
        mesh=_mesh(),
        scratch_types=[
            pltpu.VMEM_SHARED((NP, HH), jnp.float32),
            pltpu.VMEM((CHA, ED), jnp.float32),
            pltpu.VMEM((CHA, ED), jnp.float32),
            pltpu.VMEM((CHA, HH), jnp.float32),
            pltpu.VMEM((CHA, HH), jnp.float32),
            pltpu.VMEM((CHA,), jnp.int32),
            pltpu.VMEM((CHA,), jnp.int32),
            pltpu.SemaphoreType.DMA,
            pltpu.SemaphoreType.DMA,
        ],
    )
    return f(edge_attr, dst, zeros_r128)


def _kc_body(em_hbm, eml_hbm, dst_hbm, s_hbm, x_hbm, acc,
             eb0, eb1, ib0, ib1, tbuf, sc0, sc1):
    c = lax.axis_index("c")
    s = lax.axis_index("s")
    col0 = c * HH
    r0 = s * RT
    pltpu.sync_copy(eml_hbm.at[pl.ds(r0, RT), pl.ds(col0, HH)],
                    acc.at[pl.ds(r0, RT)])
    plsc.subcore_barrier()

    base = s * ET
    ebs = (eb0, eb1)
    ibs = (ib0, ib1)
    scs = (sc0, sc1)

    def pair(i, carry):
        for b in range(2):
            j = 2 * i + b
            e0 = base + j * CH

            @pl.when(j >= 2)
            def _():
                pltpu.make_async_copy(ebs[b], acc.at[ibs[b]], scs[b]).wait()

            pltpu.sync_copy(dst_hbm.at[pl.ds(e0, CH)], ibs[b])
            pltpu.sync_copy(em_hbm.at[pl.ds(e0, CH), pl.ds(col0, HH)], ebs[b])
            pltpu.async_copy(ebs[b], acc.at[ibs[b]], scs[b], add=True)
        return carry

    lax.fori_loop(0, NCH // 2, pair, 0)
    jt = NCH - 1
    e0 = base + jt * CH
    pltpu.make_async_copy(eb0, acc.at[ib0], sc0).wait()
    pltpu.sync_copy(dst_hbm.at[pl.ds(e0, CH)], ib0)
    pltpu.sync_copy(em_hbm.at[pl.ds(e0, CH), pl.ds(col0, HH)], eb0)
    pltpu.async_copy(eb0, acc.at[ib0], sc0, add=True)
    pltpu.make_async_copy(eb0, acc.at[ib0], sc0).wait()
    pltpu.make_async_copy(eb1, acc.at[ib1], sc1).wait()
    plsc.subcore_barrier()

    def out_chunk(j, carry):
        rr = r0 + j * RC
        pltpu.sync_copy(acc.at[pl.ds(rr, RC)], tbuf)
        pltpu.sync_copy(tbuf, s_hbm.at[pl.ds(rr, RC), pl.ds(col0, HH)])

        def relu_row(i, carry2):
            for g in range(HH // 16):
                v = tbuf[i, pl.ds(g * 16, 16)]
                tbuf[i, pl.ds(g * 16, 16)] = jnp.maximum(v, 0.0)
            return carry2

        lax.fori_loop(0, RC, relu_row, 0)
        pltpu.sync_copy(tbuf, x_hbm.at[pl.ds(c * NP + rr, RC), :])
        return carry

    lax.fori_loop(0, NRC, out_chunk, 0)


def _run_kc(em, eml, dst):
    f = pl.kernel(
        _kc_body,
        out_type=[jax.ShapeDtypeStruct((NP, HID), jnp.float32),
                  jax.ShapeDtypeStruct((2 * NP, HH), jnp.float32)],
        mesh=_mesh(),
        scratch_types=[
            pltpu.VMEM_SHARED((NP, HH), jnp.float32),
            pltpu.VMEM((CH, HH), jnp.float32),
            pltpu.VMEM((CH, HH), jnp.float32),
            pltpu.VMEM((CH,), jnp.int32),
            pltpu.VMEM((CH,), jnp.int32),
            pltpu.VMEM((RC, HH), jnp.float32),
            pltpu.SemaphoreType.DMA,
            pltpu.SemaphoreType.DMA,
        ],
    )
    return f(em, eml, dst)


def _kd_body(xprev_hbm, s_hbm, src0_hbm, src1_hbm, dst_hbm, z_hbm, xnext_hbm,
             acc, gb0, gb1, sib0, sib1, dib0, dib1, tbuf, tbuf2,
             sg0, sg1, sc0, sc1):
    c = lax.axis_index("c")
    s = lax.axis_index("s")
    col0 = c * HH
    r0 = s * RT
    roff = c * NP
    pltpu.sync_copy(z_hbm, acc.at[pl.ds(r0, RT)])
    plsc.subcore_barrier()

    base = s * ET
    gbs = (gb0, gb1)
    sibs = (sib0, sib1)
    dibs = (dib0, dib1)
    sgs = (sg0, sg1)
    scs = (sc0, sc1)

    def do_chunk(j, b, guarded):
        e0 = base + j * CH
        if guarded:
            @pl.when(j >= 2)
            def _():
                pltpu.make_async_copy(gbs[b], acc.at[dibs[b]], scs[b]).wait()
        else:
            pltpu.make_async_copy(gbs[b], acc.at[dibs[b]], scs[b]).wait()
        @pl.when(c == 0)
        def _():
            pltpu.sync_copy(src0_hbm.at[pl.ds(e0, CH)], sibs[b])

        @pl.when(c == 1)
        def _():
            pltpu.sync_copy(src1_hbm.at[pl.ds(e0, CH)], sibs[b])

        pltpu.sync_copy(dst_hbm.at[pl.ds(e0, CH)], dibs[b])
        pltpu.async_copy(xprev_hbm.at[sibs[b]], gbs[b], sgs[b]).wait()
        pltpu.async_copy(gbs[b], acc.at[dibs[b]], scs[b], add=True)

    def pair(i, carry):
        for b in range(2):
            do_chunk(2 * i + b, b, True)
        return carry

    lax.fori_loop(0, NCH // 2, pair, 0)
    do_chunk(NCH - 1, 0, False)
    pltpu.make_async_copy(gb0, acc.at[dib0], sc0).wait()
    pltpu.make_async_copy(gb1, acc.at[dib1], sc1).wait()
    plsc.subcore_barrier()

    def out_chunk(j, carry):
        rr = r0 + j * RC
        pltpu.sync_copy(acc.at[pl.ds(rr, RC)], tbuf)
        pltpu.sync_copy(s_hbm.at[pl.ds(rr, RC), pl.ds(col0, HH)], tbuf2)

        def add_row(i, carry2):
            for g in range(HH // 16):
                sl = pl.ds(g * 16, 16)
                tbuf[i, sl] = tbuf[i, sl] + tbuf2[i, sl]
            return carry2

        lax.fori_loop(0, RC, add_row, 0)
        pltpu.sync_copy(xprev_hbm.at[pl.ds(roff + rr, RC), :], tbuf2)

        def fuse_row(i, carry2):
            for g in range(HH // 16):
                sl = pl.ds(g * 16, 16)
                v = tbuf[i, sl] + tbuf2[i, sl]
                tbuf[i, sl] = jnp.maximum(v, 0.0)
            return carry2

        lax.fori_loop(0, RC, fuse_row, 0)
        pltpu.sync_copy(tbuf, xnext_hbm.at[pl.ds(roff + rr, RC), :])
        return carry

    lax.fori_loop(0, NRC, out_chunk, 0)


def _run_kd(xprev, s_arr, src0, src1, dst, zeros_r128):
    f = pl.kernel(
        _kd_body,
        out_type=jax.ShapeDtypeStruct((2 * NP, HH), jnp.float32),
        mesh=_mesh(),
        scratch_types=[
            pltpu.VMEM_SHARED((NP, HH), jnp.float32),
            pltpu.VMEM((CH, HH), jnp.float32),
            pltpu.VMEM((CH, HH), jnp.float32),
            pltpu.VMEM((CH,), jnp.int32),
            pltpu.VMEM((CH,), jnp.int32),
            pltpu.VMEM((CH,), jnp.int32),
            pltpu.VMEM((CH,), jnp.int32),
            pltpu.VMEM((RC, HH), jnp.float32),
            pltpu.VMEM((RC, HH), jnp.float32),
            pltpu.SemaphoreType.DMA,
            pltpu.SemaphoreType.DMA,
            pltpu.SemaphoreType.DMA,
            pltpu.SemaphoreType.DMA,
        ],
    )
    return f(xprev, s_arr, src0, src1, dst, zeros_r128)


def _kb1_body(attr_ref, w1_ref, b1_ref, w2_ref, b2_ref, w3_ref, b3_ref,
              out_ref):
    a = attr_ref[...]
    h = jnp.dot(a, w1_ref[...], preferred_element_type=jnp.float32)
    h = jnp.maximum(h + b1_ref[...], 0.0)
    h = jnp.dot(h, w2_ref[...], preferred_element_type=jnp.float32)
    h = jnp.maximum(h + b2_ref[...], 0.0)
    h = jnp.dot(h, w3_ref[...], preferred_element_type=jnp.float32)
    out_ref[...] = h + b3_ref[...]


def _run_kb1(edge_attr, w1, b1, w2, b2, w3, b3):
    blk = 1280
    nblk = NE // blk
    full = lambda shape: pl.BlockSpec(shape, lambda i: (0, 0))
    return pl.pallas_call(
        _kb1_body,
        grid=(nblk,),
        in_specs=[
            pl.BlockSpec((blk, ED), lambda i: (i, 0)),
            full((ED, HID)), full((1, HID)),
            full((HID, HID)), full((1, HID)),
            full((HID, HID)), full((1, HID)),
        ],
        out_specs=pl.BlockSpec((blk, HID), lambda i: (i, 0)),
        out_shape=jax.ShapeDtypeStruct((NE, HID), jnp.float32),
    )(edge_attr, w1, b1, w2, b2, w3, b3)


def _kb2_body(p0_ref, p1_ref,
              w1_ref, b1_ref, w2_ref, b2_ref, w3_ref, b3_ref, out_ref):
    psum = p0_ref[...] + p1_ref[...]
    attr_sum = psum[:, 0:ED]
    deg = psum[:, ED:ED + 1]
    la = attr_sum / jnp.maximum(deg, 1.0)
    h = jnp.dot(la, w1_ref[...], preferred_element_type=jnp.float32)
    h = jnp.maximum(h + b1_ref[...], 0.0)
    h = jnp.dot(h, w2_ref[...], preferred_element_type=jnp.float32)
    h = jnp.maximum(h + b2_ref[...], 0.0)
    h = jnp.dot(h, w3_ref[...], preferred_element_type=jnp.float32)
    out_ref[...] = h + b3_ref[...]


def _run_kb2(pab, w1, b1, w2, b2, w3, b3):
    blk = 1024
    nblk = NP // blk
    full = lambda shape: pl.BlockSpec(shape, lambda i: (0, 0))
    return pl.pallas_call(
        _kb2_body,
        grid=(nblk,),
        in_specs=[
            pl.BlockSpec((blk, HH), lambda i: (i, 0)),
            pl.BlockSpec((blk, HH), lambda i: (i + nblk, 0)),
            full((ED, HID)), full((1, HID)),
            full((HID, HID)), full((1, HID)),
            full((HID, HID)), full((1, HID)),
        ],
        out_specs=pl.BlockSpec((blk, HID), lambda i: (i, 0)),
        out_shape=jax.ShapeDtypeStruct((NP, HID), jnp.float32),
    )(pab, pab, w1, b1, w2, b2, w3, b3)


def _ke_body(xa_ref, xb_ref, batch_ref, wc1_ref, bc1_ref, wc2_ref, bc2_ref,
             out_ref, g_scr):
    i = pl.program_id(0)
    nsteps = pl.num_programs(0)

    @pl.when(i == 0)
    def _():
        g_scr[...] = jnp.full((NG, HID), NEG, jnp.float32)

    b = batch_ref[...]
    xa = xa_ref[...]
    xb = xb_ref[...]
    for j in range(NG):
        m = b == j
        va = jnp.max(jnp.where(m, xa, NEG), axis=0, keepdims=True)
        vb = jnp.max(jnp.where(m, xb, NEG), axis=0, keepdims=True)
        g_scr[j:j + 1, 0:HH] = jnp.maximum(g_scr[j:j + 1, 0:HH], va)
        g_scr[j:j + 1, HH:HID] = jnp.maximum(g_scr[j:j + 1, HH:HID], vb)

    @pl.when(i == nsteps - 1)
    def _():
        g = g_scr[...]
        h = jnp.dot(g, wc1_ref[...], preferred_element_type=jnp.float32)
        h = jnp.maximum(h + bc1_ref[...], 0.0)
        h = jnp.dot(h, wc2_ref[...], preferred_element_type=jnp.float32)
        out_ref[...] = h + bc2_ref[...]


def _run_ke(x3, batch2, wc1, bc1, wc2, bc2):
    blk = 1024
    nblk = NP // blk
    full = lambda shape: pl.BlockSpec(shape, lambda i: (0, 0))
    ncls = wc2.shape[1]
    return pl.pallas_call(
        _ke_body,
        grid=(nblk,),
        in_specs=[
            pl.BlockSpec((blk, HH), lambda i: (i, 0)),
            pl.BlockSpec((blk, HH), lambda i: (i + nblk, 0)),
            pl.BlockSpec((blk, 1), lambda i: (i, 0)),
            full((HID, HID)), full((1, HID)),
            full((HID, ncls)), full((1, ncls)),
        ],
        out_specs=pl.BlockSpec((NG, ncls), lambda i: (0, 0)),
        out_shape=jax.ShapeDtypeStruct((NG, ncls), jnp.float32),
        scratch_shapes=[pltpu.VMEM((NG, HID), jnp.float32)],
    )(x3, x3, batch2, wc1, bc1, wc2, bc2)


def kernel(edge_attr, edge_index, batch, W1, b1, W2, b2, W3, b3,
           Wc1, bc1, Wc2, bc2):
    src = edge_index[0].astype(jnp.int32)
    dst = edge_index[1].astype(jnp.int32)
    src1 = src + NP
    batch_pad = jnp.concatenate(
        [batch.astype(jnp.int32), jnp.full((NP - NN,), NG, jnp.int32)])
    batch2 = batch_pad.reshape(NP, 1)
    zeros_r128 = jnp.zeros((RT, HH), jnp.float32)
    b1r = b1.reshape(1, HID)
    b2r = b2.reshape(1, HID)
    b3r = b3.reshape(1, HID)
    bc1r = bc1.reshape(1, HID)
    bc2r = bc2.reshape(1, -1)

    pab = _run_ka(edge_attr, dst, zeros_r128)
    em = _run_kb1(edge_attr, W1, b1r, W2, b2r, W3, b3r)
    eml = _run_kb2(pab, W1, b1r, W2, b2r, W3, b3r)
    s_arr, x1 = _run_kc(em, eml, dst)
    x2 = _run_kd(x1, s_arr, src, src1, dst, zeros_r128)
    x3 = _run_kd(x2, s_arr, src, src1, dst, zeros_r128)
    return _run_ke(x3, batch2, Wc1, bc1r, Wc2, bc2r)


# speedup vs baseline: 6.7822x; 1.1433x over previous
import functools

import jax
import jax.numpy as jnp
from jax import lax
from jax.experimental import pallas as pl
from jax.experimental.pallas import tpu as pltpu
from jax.experimental.pallas import tpu_sc as plsc

NN = 10000
NP = 10240
NE = 160000
ED = 16
HID = 256
HH = 128
NG = 16
NC = 2
NS = 16

EA = NE // (NC * NS)
CHA = 40
NCHA = EA // CHA

ET = NE // NS
CH = 128
NCHF = ET // CH
CHT = ET - NCHF * CH

RT = NP // NS
RC = 32
NRC = RT // RC

def _mesh():
    return plsc.VectorSubcoreMesh(core_axis_name="c", subcore_axis_name="s",
                                  num_cores=NC, num_subcores=NS)


NEG = -3.4e38


def _ka_body(attr_hbm, dst_hbm, z_hbm, pab_hbm, acc,
             ab0, ab1, sb0, sb1, ib0, ib1, sc0, sc1):
    c = lax.axis_index("c")
    s = lax.axis_index("s")
    r0 = s * RT
    pltpu.sync_copy(z_hbm, acc.at[pl.ds(r0, RT)])

    ones = jnp.ones((16,), jnp.float32)
    abs_ = (ab0, ab1)
    sbs = (sb0, sb1)
    ibs = (ib0, ib1)
    scs = (sc0, sc1)

    def preset(i, carry):
        for b in range(2):
            sbs[b][i, pl.ds(16, 16)] = ones
        return carry

    lax.fori_loop(0, CHA, preset, 0)
    plsc.subcore_barrier()

    base = (c * NS + s) * EA

    def do_chunk(j, b, guarded):
        e0 = base + j * CHA
        if guarded:
            @pl.when(j >= 2)
            def _():
                pltpu.make_async_copy(sbs[b], acc.at[ibs[b]], scs[b]).wait()
        else:
            pltpu.make_async_copy(sbs[b], acc.at[ibs[b]], scs[b]).wait()
        pltpu.sync_copy(dst_hbm.at[pl.ds(e0, CHA)], ibs[b])
        pltpu.sync_copy(attr_hbm.at[pl.ds(e0, CHA), :], abs_[b])
        for k in range(CHA):
            sbs[b][k, pl.ds(0, 16)] = abs_[b][k, :]
        pltpu.async_copy(sbs[b], acc.at[ibs[b]], scs[b], add=True)

    def pair(i, carry):
        for b in range(2):
            do_chunk(2 * i + b, b, True)
        return carry

    lax.fori_loop(0, NCHA // 2, pair, 0)
    do_chunk(NCHA - 1, 0, False)
    pltpu.make_async_copy(sb0, acc.at[ib0], sc0).wait()
    pltpu.make_async_copy(sb1, acc.at[ib1], sc1).wait()
    plsc.subcore_barrier()
    pltpu.sync_copy(acc.at[pl.ds(r0, RT)],
                    pab_hbm.at[pl.ds(c * NP + r0, RT), :])


def _run_ka(edge_attr, dst, zeros_r128):
    f = pl.kernel(
        _ka_body,
        out_type=jax.ShapeDtypeStruct((2 * NP, HH), jnp.float32),
        mesh=_mesh(),
        scratch_types=[
            pltpu.VMEM_SHARED((NP, HH), jnp.float32),
            pltpu.VMEM((CHA, ED), jnp.float32),
            pltpu.VMEM((CHA, ED), jnp.float32),
            pltpu.VMEM((CHA, HH), jnp.float32),
            pltpu.VMEM((CHA, HH), jnp.float32),
            pltpu.VMEM((CHA,), jnp.int32),
            pltpu.VMEM((CHA,), jnp.int32),
            pltpu.SemaphoreType.DMA,
            pltpu.SemaphoreType.DMA,
        ],
    )
    return f(edge_attr, dst, zeros_r128)


def _kc_body(em_hbm, eml_hbm, dst_hbm, s_hbm, x_hbm, acc,
             eb0, eb1, ib0, ib1, ebt, ibt, tbuf, sc0, sc1, sct):
    c = lax.axis_index("c")
    s = lax.axis_index("s")
    col0 = c * HH
    r0 = s * RT
    pltpu.sync_copy(eml_hbm.at[pl.ds(r0, RT), pl.ds(col0, HH)],
                    acc.at[pl.ds(r0, RT)])
    plsc.subcore_barrier()

    base = s * ET
    ebs = (eb0, eb1)
    ibs = (ib0, ib1)
    scs = (sc0, sc1)

    def pair(i, carry):
        for b in range(2):
            j = 2 * i + b
            e0 = base + j * CH

            @pl.when(j >= 2)
            def _():
                pltpu.make_async_copy(ebs[b], acc.at[ibs[b]], scs[b]).wait()

            pltpu.sync_copy(dst_hbm.at[pl.ds(e0, CH)], ibs[b])
            pltpu.sync_copy(em_hbm.at[pl.ds(e0, CH), pl.ds(col0, HH)], ebs[b])
            pltpu.async_copy(ebs[b], acc.at[ibs[b]], scs[b], add=True)
        return carry

    lax.fori_loop(0, NCHF // 2, pair, 0)
    e0 = base + NCHF * CH
    pltpu.sync_copy(dst_hbm.at[pl.ds(e0, CHT)], ibt)
    pltpu.sync_copy(em_hbm.at[pl.ds(e0, CHT), pl.ds(col0, HH)], ebt)
    pltpu.async_copy(ebt, acc.at[ibt], sct, add=True)
    pltpu.make_async_copy(eb0, acc.at[ib0], sc0).wait()
    pltpu.make_async_copy(eb1, acc.at[ib1], sc1).wait()
    pltpu.make_async_copy(ebt, acc.at[ibt], sct).wait()
    plsc.subcore_barrier()

    def out_chunk(j, carry):
        rr = r0 + j * RC
        pltpu.sync_copy(acc.at[pl.ds(rr, RC)], tbuf)
        pltpu.sync_copy(tbuf, s_hbm.at[pl.ds(rr, RC), pl.ds(col0, HH)])

        def relu_row(i, carry2):
            for g in range(HH // 16):
                v = tbuf[i, pl.ds(g * 16, 16)]
                tbuf[i, pl.ds(g * 16, 16)] = jnp.maximum(v, 0.0)
            return carry2

        lax.fori_loop(0, RC, relu_row, 0)
        pltpu.sync_copy(tbuf, x_hbm.at[pl.ds(c * NP + rr, RC), :])
        return carry

    lax.fori_loop(0, NRC, out_chunk, 0)


def _run_kc(em, eml, dst):
    f = pl.kernel(
        _kc_body,
        out_type=[jax.ShapeDtypeStruct((NP, HID), jnp.float32),
                  jax.ShapeDtypeStruct((2 * NP, HH), jnp.float32)],
        mesh=_mesh(),
        scratch_types=[
            pltpu.VMEM_SHARED((NP, HH), jnp.float32),
            pltpu.VMEM((CH, HH), jnp.float32),
            pltpu.VMEM((CH, HH), jnp.float32),
            pltpu.VMEM((CH,), jnp.int32),
            pltpu.VMEM((CH,), jnp.int32),
            pltpu.VMEM((CHT, HH), jnp.float32),
            pltpu.VMEM((CHT,), jnp.int32),
            pltpu.VMEM((RC, HH), jnp.float32),
            pltpu.SemaphoreType.DMA,
            pltpu.SemaphoreType.DMA,
            pltpu.SemaphoreType.DMA,
        ],
    )
    return f(em, eml, dst)


def _kd_body(xprev_hbm, s_hbm, src0_hbm, src1_hbm, dst_hbm, z_hbm, xnext_hbm,
             acc, gb0, gb1, sib0, sib1, dib0, dib1, gbt, sibt, dibt,
             tbuf, tbuf2, sg0, sg1, sc0, sc1, sgt, sct):
    c = lax.axis_index("c")
    s = lax.axis_index("s")
    col0 = c * HH
    r0 = s * RT
    roff = c * NP
    pltpu.sync_copy(z_hbm, acc.at[pl.ds(r0, RT)])
    plsc.subcore_barrier()

    base = s * ET
    gbs = (gb0, gb1)
    sibs = (sib0, sib1)
    dibs = (dib0, dib1)
    sgs = (sg0, sg1)
    scs = (sc0, sc1)

    def do_chunk(j, b, guarded):
        e0 = base + j * CH
        if guarded:
            @pl.when(j >= 2)
            def _():
                pltpu.make_async_copy(gbs[b], acc.at[dibs[b]], scs[b]).wait()
        else:
            pltpu.make_async_copy(gbs[b], acc.at[dibs[b]], scs[b]).wait()
        @pl.when(c == 0)
        def _():
            pltpu.sync_copy(src0_hbm.at[pl.ds(e0, CH)], sibs[b])

        @pl.when(c == 1)
        def _():
            pltpu.sync_copy(src1_hbm.at[pl.ds(e0, CH)], sibs[b])

        pltpu.sync_copy(dst_hbm.at[pl.ds(e0, CH)], dibs[b])
        pltpu.async_copy(xprev_hbm.at[sibs[b]], gbs[b], sgs[b]).wait()
        pltpu.async_copy(gbs[b], acc.at[dibs[b]], scs[b], add=True)

    def pair(i, carry):
        for b in range(2):
            do_chunk(2 * i + b, b, True)
        return carry

    lax.fori_loop(0, NCHF // 2, pair, 0)
    e0 = base + NCHF * CH

    @pl.when(c == 0)
    def _():
        pltpu.sync_copy(src0_hbm.at[pl.ds(e0, CHT)], sibt)

    @pl.when(c == 1)
    def _():
        pltpu.sync_copy(src1_hbm.at[pl.ds(e0, CHT)], sibt)

    pltpu.sync_copy(dst_hbm.at[pl.ds(e0, CHT)], dibt)
    pltpu.async_copy(xprev_hbm.at[sibt], gbt, sgt).wait()
    pltpu.async_copy(gbt, acc.at[dibt], sct, add=True)
    pltpu.make_async_copy(gb0, acc.at[dib0], sc0).wait()
    pltpu.make_async_copy(gb1, acc.at[dib1], sc1).wait()
    pltpu.make_async_copy(gbt, acc.at[dibt], sct).wait()
    plsc.subcore_barrier()

    def out_chunk(j, carry):
        rr = r0 + j * RC
        pltpu.sync_copy(acc.at[pl.ds(rr, RC)], tbuf)
        pltpu.sync_copy(s_hbm.at[pl.ds(rr, RC), pl.ds(col0, HH)], tbuf2)

        def add_row(i, carry2):
            for g in range(HH // 16):
                sl = pl.ds(g * 16, 16)
                tbuf[i, sl] = tbuf[i, sl] + tbuf2[i, sl]
            return carry2

        lax.fori_loop(0, RC, add_row, 0)
        pltpu.sync_copy(xprev_hbm.at[pl.ds(roff + rr, RC), :], tbuf2)

        def fuse_row(i, carry2):
            for g in range(HH // 16):
                sl = pl.ds(g * 16, 16)
                v = tbuf[i, sl] + tbuf2[i, sl]
                tbuf[i, sl] = jnp.maximum(v, 0.0)
            return carry2

        lax.fori_loop(0, RC, fuse_row, 0)
        pltpu.sync_copy(tbuf, xnext_hbm.at[pl.ds(roff + rr, RC), :])
        return carry

    lax.fori_loop(0, NRC, out_chunk, 0)


def _run_kd(xprev, s_arr, src0, src1, dst, zeros_r128):
    f = pl.kernel(
        _kd_body,
        out_type=jax.ShapeDtypeStruct((2 * NP, HH), jnp.float32),
        mesh=_mesh(),
        scratch_types=[
            pltpu.VMEM_SHARED((NP, HH), jnp.float32),
            pltpu.VMEM((CH, HH), jnp.float32),
            pltpu.VMEM((CH, HH), jnp.float32),
            pltpu.VMEM((CH,), jnp.int32),
            pltpu.VMEM((CH,), jnp.int32),
            pltpu.VMEM((CH,), jnp.int32),
            pltpu.VMEM((CH,), jnp.int32),
            pltpu.VMEM((CHT, HH), jnp.float32),
            pltpu.VMEM((CHT,), jnp.int32),
            pltpu.VMEM((CHT,), jnp.int32),
            pltpu.VMEM((RC, HH), jnp.float32),
            pltpu.VMEM((RC, HH), jnp.float32),
            pltpu.SemaphoreType.DMA,
            pltpu.SemaphoreType.DMA,
            pltpu.SemaphoreType.DMA,
            pltpu.SemaphoreType.DMA,
            pltpu.SemaphoreType.DMA,
            pltpu.SemaphoreType.DMA,
        ],
    )
    return f(xprev, s_arr, src0, src1, dst, zeros_r128)


def _kb1_body(attr_ref, w1_ref, b1_ref, w2_ref, b2_ref, w3_ref, b3_ref,
              out_ref):
    a = attr_ref[...]
    h = jnp.dot(a, w1_ref[...], preferred_element_type=jnp.float32)
    h = jnp.maximum(h + b1_ref[...], 0.0)
    h = jnp.dot(h, w2_ref[...], preferred_element_type=jnp.float32)
    h = jnp.maximum(h + b2_ref[...], 0.0)
    h = jnp.dot(h, w3_ref[...], preferred_element_type=jnp.float32)
    out_ref[...] = h + b3_ref[...]


def _run_kb1(edge_attr, w1, b1, w2, b2, w3, b3):
    blk = 1280
    nblk = NE // blk
    full = lambda shape: pl.BlockSpec(shape, lambda i: (0, 0))
    return pl.pallas_call(
        _kb1_body,
        grid=(nblk,),
        in_specs=[
            pl.BlockSpec((blk, ED), lambda i: (i, 0)),
            full((ED, HID)), full((1, HID)),
            full((HID, HID)), full((1, HID)),
            full((HID, HID)), full((1, HID)),
        ],
        out_specs=pl.BlockSpec((blk, HID), lambda i: (i, 0)),
        out_shape=jax.ShapeDtypeStruct((NE, HID), jnp.float32),
    )(edge_attr, w1, b1, w2, b2, w3, b3)


def _kb2_body(p0_ref, p1_ref,
              w1_ref, b1_ref, w2_ref, b2_ref, w3_ref, b3_ref, out_ref):
    psum = p0_ref[...] + p1_ref[...]
    attr_sum = psum[:, 0:ED]
    deg = psum[:, ED:ED + 1]
    la = attr_sum / jnp.maximum(deg, 1.0)
    h = jnp.dot(la, w1_ref[...], preferred_element_type=jnp.float32)
    h = jnp.maximum(h + b1_ref[...], 0.0)
    h = jnp.dot(h, w2_ref[...], preferred_element_type=jnp.float32)
    h = jnp.maximum(h + b2_ref[...], 0.0)
    h = jnp.dot(h, w3_ref[...], preferred_element_type=jnp.float32)
    out_ref[...] = h + b3_ref[...]


def _run_kb2(pab, w1, b1, w2, b2, w3, b3):
    blk = 1024
    nblk = NP // blk
    full = lambda shape: pl.BlockSpec(shape, lambda i: (0, 0))
    return pl.pallas_call(
        _kb2_body,
        grid=(nblk,),
        in_specs=[
            pl.BlockSpec((blk, HH), lambda i: (i, 0)),
            pl.BlockSpec((blk, HH), lambda i: (i + nblk, 0)),
            full((ED, HID)), full((1, HID)),
            full((HID, HID)), full((1, HID)),
            full((HID, HID)), full((1, HID)),
        ],
        out_specs=pl.BlockSpec((blk, HID), lambda i: (i, 0)),
        out_shape=jax.ShapeDtypeStruct((NP, HID), jnp.float32),
    )(pab, pab, w1, b1, w2, b2, w3, b3)


def _ke_body(xa_ref, xb_ref, batch_ref, wc1_ref, bc1_ref, wc2_ref, bc2_ref,
             out_ref, g_scr):
    i = pl.program_id(0)
    nsteps = pl.num_programs(0)

    @pl.when(i == 0)
    def _():
        g_scr[...] = jnp.full((NG, HID), NEG, jnp.float32)

    b = batch_ref[...]
    xa = xa_ref[...]
    xb = xb_ref[...]
    for j in range(NG):
        m = b == j
        va = jnp.max(jnp.where(m, xa, NEG), axis=0, keepdims=True)
        vb = jnp.max(jnp.where(m, xb, NEG), axis=0, keepdims=True)
        g_scr[j:j + 1, 0:HH] = jnp.maximum(g_scr[j:j + 1, 0:HH], va)
        g_scr[j:j + 1, HH:HID] = jnp.maximum(g_scr[j:j + 1, HH:HID], vb)

    @pl.when(i == nsteps - 1)
    def _():
        g = g_scr[...]
        h = jnp.dot(g, wc1_ref[...], preferred_element_type=jnp.float32)
        h = jnp.maximum(h + bc1_ref[...], 0.0)
        h = jnp.dot(h, wc2_ref[...], preferred_element_type=jnp.float32)
        out_ref[...] = h + bc2_ref[...]


def _run_ke(x3, batch2, wc1, bc1, wc2, bc2):
    blk = 1024
    nblk = NP // blk
    full = lambda shape: pl.BlockSpec(shape, lambda i: (0, 0))
    ncls = wc2.shape[1]
    return pl.pallas_call(
        _ke_body,
        grid=(nblk,),
        in_specs=[
            pl.BlockSpec((blk, HH), lambda i: (i, 0)),
            pl.BlockSpec((blk, HH), lambda i: (i + nblk, 0)),
            pl.BlockSpec((blk, 1), lambda i: (i, 0)),
            full((HID, HID)), full((1, HID)),
            full((HID, ncls)), full((1, ncls)),
        ],
        out_specs=pl.BlockSpec((NG, ncls), lambda i: (0, 0)),
        out_shape=jax.ShapeDtypeStruct((NG, ncls), jnp.float32),
        scratch_shapes=[pltpu.VMEM((NG, HID), jnp.float32)],
    )(x3, x3, batch2, wc1, bc1, wc2, bc2)


def kernel(edge_attr, edge_index, batch, W1, b1, W2, b2, W3, b3,
           Wc1, bc1, Wc2, bc2):
    src = edge_index[0].astype(jnp.int32)
    dst = edge_index[1].astype(jnp.int32)
    src1 = src + NP
    batch_pad = jnp.concatenate(
        [batch.astype(jnp.int32), jnp.full((NP - NN,), NG, jnp.int32)])
    batch2 = batch_pad.reshape(NP, 1)
    zeros_r128 = jnp.zeros((RT, HH), jnp.float32)
    b1r = b1.reshape(1, HID)
    b2r = b2.reshape(1, HID)
    b3r = b3.reshape(1, HID)
    bc1r = bc1.reshape(1, HID)
    bc2r = bc2.reshape(1, -1)

    pab = _run_ka(edge_attr, dst, zeros_r128)
    em = _run_kb1(edge_attr, W1, b1r, W2, b2r, W3, b3r)
    eml = _run_kb2(pab, W1, b1r, W2, b2r, W3, b3r)
    s_arr, x1 = _run_kc(em, eml, dst)
    x2 = _run_kd(x1, s_arr, src, src1, dst, zeros_r128)
    x3 = _run_kd(x2, s_arr, src, src1, dst, zeros_r128)
    return _run_ke(x3, batch2, Wc1, bc1r, Wc2, bc2r)


# speedup vs baseline: 7.1019x; 1.0471x over previous
import functools

import jax
import jax.numpy as jnp
from jax import lax
from jax.experimental import pallas as pl
from jax.experimental.pallas import tpu as pltpu
from jax.experimental.pallas import tpu_sc as plsc

NN = 10000
NP = 10240
NE = 160000
ED = 16
HID = 256
HH = 128
NG = 16
NC = 2
NS = 16

EA = NE // (NC * NS)
CHA = 80
NCHAF = EA // CHA
CHAT = EA - NCHAF * CHA

ET = NE // NS
CH = 128
NCHF = ET // CH
CHT = ET - NCHF * CH

RT = NP // NS
RC = 32
NRC = RT // RC

def _mesh():
    return plsc.VectorSubcoreMesh(core_axis_name="c", subcore_axis_name="s",
                                  num_cores=NC, num_subcores=NS)


NEG = -3.4e38


def _ka_body(attr_hbm, dst_hbm, z_hbm, pab_hbm, acc,
             ab0, ab1, sb0, sb1, ib0, ib1, sc0, sc1):
    c = lax.axis_index("c")
    s = lax.axis_index("s")
    r0 = s * RT
    pltpu.sync_copy(z_hbm, acc.at[pl.ds(r0, RT)])

    ones = jnp.ones((16,), jnp.float32)
    abs_ = (ab0, ab1)
    sbs = (sb0, sb1)
    ibs = (ib0, ib1)
    scs = (sc0, sc1)

    def preset(i, carry):
        for b in range(2):
            sbs[b][i, pl.ds(16, 16)] = ones
        return carry

    lax.fori_loop(0, CHA, preset, 0)
    plsc.subcore_barrier()

    base = (c * NS + s) * EA

    def do_chunk(j, b, guarded):
        e0 = base + j * CHA
        if guarded:
            @pl.when(j >= 2)
            def _():
                pltpu.make_async_copy(sbs[b], acc.at[ibs[b]], scs[b]).wait()
        else:
            pltpu.make_async_copy(sbs[b], acc.at[ibs[b]], scs[b]).wait()
        pltpu.sync_copy(dst_hbm.at[pl.ds(e0, CHA)], ibs[b])
        pltpu.sync_copy(attr_hbm.at[pl.ds(e0, CHA), :], abs_[b])
        for k in range(CHA):
            sbs[b][k, pl.ds(0, 16)] = abs_[b][k, :]
        pltpu.async_copy(sbs[b], acc.at[ibs[b]], scs[b], add=True)

    def pair(i, carry):
        for b in range(2):
            do_chunk(2 * i + b, b, True)
        return carry

    lax.fori_loop(0, NCHAF // 2, pair, 0)
    e0 = base + NCHAF * CHA
    pltpu.make_async_copy(sb0, acc.at[ib0], sc0).wait()
    zeros16 = jnp.zeros((16,), jnp.float32)
    for k in range(CHAT, CHA):
        sb0[k, pl.ds(0, 16)] = zeros16
        sb0[k, pl.ds(16, 16)] = zeros16
    pltpu.sync_copy(dst_hbm.at[pl.ds(e0, CHAT)], ib0.at[pl.ds(0, CHAT)])
    pltpu.sync_copy(attr_hbm.at[pl.ds(e0, CHAT), :], ab0.at[pl.ds(0, CHAT)])
    for k in range(CHAT):
        sb0[k, pl.ds(0, 16)] = ab0[k, :]
    pltpu.async_copy(sb0, acc.at[ib0], sc0, add=True)
    pltpu.make_async_copy(sb0, acc.at[ib0], sc0).wait()
    pltpu.make_async_copy(sb1, acc.at[ib1], sc1).wait()
    plsc.subcore_barrier()
    pltpu.sync_copy(acc.at[pl.ds(r0, RT)],
                    pab_hbm.at[pl.ds(c * NP + r0, RT), :])


def _run_ka(edge_attr, dst, zeros_r128):
    f = pl.kernel(
        _ka_body,
        out_type=jax.ShapeDtypeStruct((2 * NP, HH), jnp.float32),
        mesh=_mesh(),
        scratch_types=[
            pltpu.VMEM_SHARED((NP, HH), jnp.float32),
            pltpu.VMEM((CHA, ED), jnp.float32),
            pltpu.VMEM((CHA, ED), jnp.float32),
            pltpu.VMEM((CHA, HH), jnp.float32),
            pltpu.VMEM((CHA, HH), jnp.float32),
            pltpu.VMEM((CHA,), jnp.int32),
            pltpu.VMEM((CHA,), jnp.int32),
            pltpu.SemaphoreType.DMA,
            pltpu.SemaphoreType.DMA,
        ],
    )
    return f(edge_attr, dst, zeros_r128)


def _kc_body(em_hbm, eml_hbm, dst_hbm, s_hbm, x_hbm, acc,
             eb0, eb1, ib0, ib1, ebt, ibt, tbuf, sc0, sc1, sct):
    c = lax.axis_index("c")
    s = lax.axis_index("s")
    col0 = c * HH
    r0 = s * RT
    pltpu.sync_copy(eml_hbm.at[pl.ds(r0, RT), pl.ds(col0, HH)],
                    acc.at[pl.ds(r0, RT)])
    plsc.subcore_barrier()

    base = s * ET
    ebs = (eb0, eb1)
    ibs = (ib0, ib1)
    scs = (sc0, sc1)

    def pair(i, carry):
        for b in range(2):
            j = 2 * i + b
            e0 = base + j * CH

            @pl.when(j >= 2)
            def _():
                pltpu.make_async_copy(ebs[b], acc.at[ibs[b]], scs[b]).wait()

            pltpu.sync_copy(dst_hbm.at[pl.ds(e0, CH)], ibs[b])
            pltpu.sync_copy(em_hbm.at[pl.ds(e0, CH), pl.ds(col0, HH)], ebs[b])
            pltpu.async_copy(ebs[b], acc.at[ibs[b]], scs[b], add=True)
        return carry

    lax.fori_loop(0, NCHF // 2, pair, 0)
    e0 = base + NCHF * CH
    pltpu.sync_copy(dst_hbm.at[pl.ds(e0, CHT)], ibt)
    pltpu.sync_copy(em_hbm.at[pl.ds(e0, CHT), pl.ds(col0, HH)], ebt)
    pltpu.async_copy(ebt, acc.at[ibt], sct, add=True)
    pltpu.make_async_copy(eb0, acc.at[ib0], sc0).wait()
    pltpu.make_async_copy(eb1, acc.at[ib1], sc1).wait()
    pltpu.make_async_copy(ebt, acc.at[ibt], sct).wait()
    plsc.subcore_barrier()

    def out_chunk(j, carry):
        rr = r0 + j * RC
        pltpu.sync_copy(acc.at[pl.ds(rr, RC)], tbuf)
        pltpu.sync_copy(tbuf, s_hbm.at[pl.ds(rr, RC), pl.ds(col0, HH)])

        def relu_row(i, carry2):
            for g in range(HH // 16):
                v = tbuf[i, pl.ds(g * 16, 16)]
                tbuf[i, pl.ds(g * 16, 16)] = jnp.maximum(v, 0.0)
            return carry2

        lax.fori_loop(0, RC, relu_row, 0)
        pltpu.sync_copy(tbuf, x_hbm.at[pl.ds(c * NP + rr, RC), :])
        return carry

    lax.fori_loop(0, NRC, out_chunk, 0)


def _run_kc(em, eml, dst):
    f = pl.kernel(
        _kc_body,
        out_type=[jax.ShapeDtypeStruct((NP, HID), jnp.float32),
                  jax.ShapeDtypeStruct((2 * NP, HH), jnp.float32)],
        mesh=_mesh(),
        scratch_types=[
            pltpu.VMEM_SHARED((NP, HH), jnp.float32),
            pltpu.VMEM((CH, HH), jnp.float32),
            pltpu.VMEM((CH, HH), jnp.float32),
            pltpu.VMEM((CH,), jnp.int32),
            pltpu.VMEM((CH,), jnp.int32),
            pltpu.VMEM((CHT, HH), jnp.float32),
            pltpu.VMEM((CHT,), jnp.int32),
            pltpu.VMEM((RC, HH), jnp.float32),
            pltpu.SemaphoreType.DMA,
            pltpu.SemaphoreType.DMA,
            pltpu.SemaphoreType.DMA,
        ],
    )
    return f(em, eml, dst)


def _kd_body(xprev_hbm, s_hbm, src0_hbm, src1_hbm, dst_hbm, z_hbm, xnext_hbm,
             acc, gb0, gb1, sib0, sib1, dib0, dib1, gbt, sibt, dibt,
             tbuf, tbuf2, sg0, sg1, sc0, sc1, sgt, sct):
    c = lax.axis_index("c")
    s = lax.axis_index("s")
    col0 = c * HH
    r0 = s * RT
    roff = c * NP
    pltpu.sync_copy(z_hbm, acc.at[pl.ds(r0, RT)])
    plsc.subcore_barrier()

    base = s * ET
    gbs = (gb0, gb1)
    sibs = (sib0, sib1)
    dibs = (dib0, dib1)
    sgs = (sg0, sg1)
    scs = (sc0, sc1)

    def do_chunk(j, b, guarded):
        e0 = base + j * CH
        if guarded:
            @pl.when(j >= 2)
            def _():
                pltpu.make_async_copy(gbs[b], acc.at[dibs[b]], scs[b]).wait()
        else:
            pltpu.make_async_copy(gbs[b], acc.at[dibs[b]], scs[b]).wait()
        @pl.when(c == 0)
        def _():
            pltpu.sync_copy(src0_hbm.at[pl.ds(e0, CH)], sibs[b])

        @pl.when(c == 1)
        def _():
            pltpu.sync_copy(src1_hbm.at[pl.ds(e0, CH)], sibs[b])

        pltpu.sync_copy(dst_hbm.at[pl.ds(e0, CH)], dibs[b])
        pltpu.async_copy(xprev_hbm.at[sibs[b]], gbs[b], sgs[b]).wait()
        pltpu.async_copy(gbs[b], acc.at[dibs[b]], scs[b], add=True)

    def pair(i, carry):
        for b in range(2):
            do_chunk(2 * i + b, b, True)
        return carry

    lax.fori_loop(0, NCHF // 2, pair, 0)
    e0 = base + NCHF * CH

    @pl.when(c == 0)
    def _():
        pltpu.sync_copy(src0_hbm.at[pl.ds(e0, CHT)], sibt)

    @pl.when(c == 1)
    def _():
        pltpu.sync_copy(src1_hbm.at[pl.ds(e0, CHT)], sibt)

    pltpu.sync_copy(dst_hbm.at[pl.ds(e0, CHT)], dibt)
    pltpu.async_copy(xprev_hbm.at[sibt], gbt, sgt).wait()
    pltpu.async_copy(gbt, acc.at[dibt], sct, add=True)
    pltpu.make_async_copy(gb0, acc.at[dib0], sc0).wait()
    pltpu.make_async_copy(gb1, acc.at[dib1], sc1).wait()
    pltpu.make_async_copy(gbt, acc.at[dibt], sct).wait()
    plsc.subcore_barrier()

    def out_chunk(j, carry):
        rr = r0 + j * RC
        pltpu.sync_copy(acc.at[pl.ds(rr, RC)], tbuf)
        pltpu.sync_copy(s_hbm.at[pl.ds(rr, RC), pl.ds(col0, HH)], tbuf2)

        def add_row(i, carry2):
            for g in range(HH // 16):
                sl = pl.ds(g * 16, 16)
                tbuf[i, sl] = tbuf[i, sl] + tbuf2[i, sl]
            return carry2

        lax.fori_loop(0, RC, add_row, 0)
        pltpu.sync_copy(xprev_hbm.at[pl.ds(roff + rr, RC), :], tbuf2)

        def fuse_row(i, carry2):
            for g in range(HH // 16):
                sl = pl.ds(g * 16, 16)
                v = tbuf[i, sl] + tbuf2[i, sl]
                tbuf[i, sl] = jnp.maximum(v, 0.0)
            return carry2

        lax.fori_loop(0, RC, fuse_row, 0)
        pltpu.sync_copy(tbuf, xnext_hbm.at[pl.ds(roff + rr, RC), :])
        return carry

    lax.fori_loop(0, NRC, out_chunk, 0)


def _run_kd(xprev, s_arr, src0, src1, dst, zeros_r128):
    f = pl.kernel(
        _kd_body,
        out_type=jax.ShapeDtypeStruct((2 * NP, HH), jnp.float32),
        mesh=_mesh(),
        scratch_types=[
            pltpu.VMEM_SHARED((NP, HH), jnp.float32),
            pltpu.VMEM((CH, HH), jnp.float32),
            pltpu.VMEM((CH, HH), jnp.float32),
            pltpu.VMEM((CH,), jnp.int32),
            pltpu.VMEM((CH,), jnp.int32),
            pltpu.VMEM((CH,), jnp.int32),
            pltpu.VMEM((CH,), jnp.int32),
            pltpu.VMEM((CHT, HH), jnp.float32),
            pltpu.VMEM((CHT,), jnp.int32),
            pltpu.VMEM((CHT,), jnp.int32),
            pltpu.VMEM((RC, HH), jnp.float32),
            pltpu.VMEM((RC, HH), jnp.float32),
            pltpu.SemaphoreType.DMA,
            pltpu.SemaphoreType.DMA,
            pltpu.SemaphoreType.DMA,
            pltpu.SemaphoreType.DMA,
            pltpu.SemaphoreType.DMA,
            pltpu.SemaphoreType.DMA,
        ],
    )
    return f(xprev, s_arr, src0, src1, dst, zeros_r128)


def _kb1_body(attr_ref, w1_ref, b1_ref, w2_ref, b2_ref, w3_ref, b3_ref,
              out_ref):
    a = attr_ref[...]
    h = jnp.dot(a, w1_ref[...], preferred_element_type=jnp.float32)
    h = jnp.maximum(h + b1_ref[...], 0.0)
    h = jnp.dot(h, w2_ref[...], preferred_element_type=jnp.float32)
    h = jnp.maximum(h + b2_ref[...], 0.0)
    h = jnp.dot(h, w3_ref[...], preferred_element_type=jnp.float32)
    out_ref[...] = h + b3_ref[...]


def _run_kb1(edge_attr, w1, b1, w2, b2, w3, b3):
    blk = 1280
    nblk = NE // blk
    full = lambda shape: pl.BlockSpec(shape, lambda i: (0, 0))
    return pl.pallas_call(
        _kb1_body,
        grid=(nblk,),
        in_specs=[
            pl.BlockSpec((blk, ED), lambda i: (i, 0)),
            full((ED, HID)), full((1, HID)),
            full((HID, HID)), full((1, HID)),
            full((HID, HID)), full((1, HID)),
        ],
        out_specs=pl.BlockSpec((blk, HID), lambda i: (i, 0)),
        out_shape=jax.ShapeDtypeStruct((NE, HID), jnp.float32),
    )(edge_attr, w1, b1, w2, b2, w3, b3)


def _kb2_body(p0_ref, p1_ref,
              w1_ref, b1_ref, w2_ref, b2_ref, w3_ref, b3_ref, out_ref):
    psum = p0_ref[...] + p1_ref[...]
    attr_sum = psum[:, 0:ED]
    deg = psum[:, ED:ED + 1]
    la = attr_sum / jnp.maximum(deg, 1.0)
    h = jnp.dot(la, w1_ref[...], preferred_element_type=jnp.float32)
    h = jnp.maximum(h + b1_ref[...], 0.0)
    h = jnp.dot(h, w2_ref[...], preferred_element_type=jnp.float32)
    h = jnp.maximum(h + b2_ref[...], 0.0)
    h = jnp.dot(h, w3_ref[...], preferred_element_type=jnp.float32)
    out_ref[...] = h + b3_ref[...]


def _run_kb2(pab, w1, b1, w2, b2, w3, b3):
    blk = 1024
    nblk = NP // blk
    full = lambda shape: pl.BlockSpec(shape, lambda i: (0, 0))
    return pl.pallas_call(
        _kb2_body,
        grid=(nblk,),
        in_specs=[
            pl.BlockSpec((blk, HH), lambda i: (i, 0)),
            pl.BlockSpec((blk, HH), lambda i: (i + nblk, 0)),
            full((ED, HID)), full((1, HID)),
            full((HID, HID)), full((1, HID)),
            full((HID, HID)), full((1, HID)),
        ],
        out_specs=pl.BlockSpec((blk, HID), lambda i: (i, 0)),
        out_shape=jax.ShapeDtypeStruct((NP, HID), jnp.float32),
    )(pab, pab, w1, b1, w2, b2, w3, b3)


def _ke_body(xa_ref, xb_ref, batch_ref, wc1_ref, bc1_ref, wc2_ref, bc2_ref,
             out_ref, g_scr):
    i = pl.program_id(0)
    nsteps = pl.num_programs(0)

    @pl.when(i == 0)
    def _():
        g_scr[...] = jnp.full((NG, HID), NEG, jnp.float32)

    b = batch_ref[...]
    xa = xa_ref[...]
    xb = xb_ref[...]
    for j in range(NG):
        m = b == j
        va = jnp.max(jnp.where(m, xa, NEG), axis=0, keepdims=True)
        vb = jnp.max(jnp.where(m, xb, NEG), axis=0, keepdims=True)
        g_scr[j:j + 1, 0:HH] = jnp.maximum(g_scr[j:j + 1, 0:HH], va)
        g_scr[j:j + 1, HH:HID] = jnp.maximum(g_scr[j:j + 1, HH:HID], vb)

    @pl.when(i == nsteps - 1)
    def _():
        g = g_scr[...]
        h = jnp.dot(g, wc1_ref[...], preferred_element_type=jnp.float32)
        h = jnp.maximum(h + bc1_ref[...], 0.0)
        h = jnp.dot(h, wc2_ref[...], preferred_element_type=jnp.float32)
        out_ref[...] = h + bc2_ref[...]


def _run_ke(x3, batch2, wc1, bc1, wc2, bc2):
    blk = 1024
    nblk = NP // blk
    full = lambda shape: pl.BlockSpec(shape, lambda i: (0, 0))
    ncls = wc2.shape[1]
    return pl.pallas_call(
        _ke_body,
        grid=(nblk,),
        in_specs=[
            pl.BlockSpec((blk, HH), lambda i: (i, 0)),
            pl.BlockSpec((blk, HH), lambda i: (i + nblk, 0)),
            pl.BlockSpec((blk, 1), lambda i: (i, 0)),
            full((HID, HID)), full((1, HID)),
            full((HID, ncls)), full((1, ncls)),
        ],
        out_specs=pl.BlockSpec((NG, ncls), lambda i: (0, 0)),
        out_shape=jax.ShapeDtypeStruct((NG, ncls), jnp.float32),
        scratch_shapes=[pltpu.VMEM((NG, HID), jnp.float32)],
    )(x3, x3, batch2, wc1, bc1, wc2, bc2)


def kernel(edge_attr, edge_index, batch, W1, b1, W2, b2, W3, b3,
           Wc1, bc1, Wc2, bc2):
    src = edge_index[0].astype(jnp.int32)
    dst = edge_index[1].astype(jnp.int32)
    src1 = src + NP
    batch_pad = jnp.concatenate(
        [batch.astype(jnp.int32), jnp.full((NP - NN,), NG, jnp.int32)])
    batch2 = batch_pad.reshape(NP, 1)
    zeros_r128 = jnp.zeros((RT, HH), jnp.float32)
    b1r = b1.reshape(1, HID)
    b2r = b2.reshape(1, HID)
    b3r = b3.reshape(1, HID)
    bc1r = bc1.reshape(1, HID)
    bc2r = bc2.reshape(1, -1)

    pab = _run_ka(edge_attr, dst, zeros_r128)
    em = _run_kb1(edge_attr, W1, b1r, W2, b2r, W3, b3r)
    eml = _run_kb2(pab, W1, b1r, W2, b2r, W3, b3r)
    s_arr, x1 = _run_kc(em, eml, dst)
    x2 = _run_kd(x1, s_arr, src, src1, dst, zeros_r128)
    x3 = _run_kd(x2, s_arr, src, src1, dst, zeros_r128)
    return _run_ke(x3, batch2, Wc1, bc1r, Wc2, bc2r)
